# Initial kernel scaffold; baseline (speedup 1.0000x reference)
import functools

import jax
import jax.numpy as jnp
from jax import lax
from jax.experimental import pallas as pl
from jax.experimental.pallas import tpu as pltpu
from jax.experimental.pallas import tpu_sc as plsc

NUM_CORES = 2
NUM_SUBCORES = 16
LANES = 16
NUM_WORKERS = NUM_CORES * NUM_SUBCORES

BATCH = 16384
EMBED = 128
BPW = BATCH // NUM_WORKERS
CHUNK = 256
NCHUNKS = BPW // CHUNK


def _dot_sigmoid_kernel(tidx_hbm, nidx_hbm, temb_hbm, nemb_hbm,
                        tb_hbm, nb_hbm, out_hbm,
                        tidx_v, nidx_v, trows_v, nrows_v,
                        tb_v, nb_v, out_v, sem):
  wid = lax.axis_index("s") * NUM_CORES + lax.axis_index("c")
  base = wid * BPW

  pltpu.sync_copy(tidx_hbm.at[pl.ds(base, BPW)], tidx_v)
  pltpu.sync_copy(nidx_hbm.at[pl.ds(base, BPW)], nidx_v)

  lane = lax.iota(jnp.int32, (LANES,), 0)

  for c in range(NCHUNKS):
    idx_t = tidx_v.at[pl.ds(c * CHUNK, CHUNK)]
    idx_n = nidx_v.at[pl.ds(c * CHUNK, CHUNK)]
    cps = (
        pltpu.async_copy(temb_hbm.at[idx_t], trows_v, sem),
        pltpu.async_copy(nemb_hbm.at[idx_n], nrows_v, sem),
        pltpu.async_copy(tb_hbm.at[idx_t], tb_v, sem),
        pltpu.async_copy(nb_hbm.at[idx_n], nb_v, sem),
    )
    for cp in cps:
      cp.wait()

    @pl.loop(0, CHUNK // LANES)
    def _(g):
      dots = jnp.zeros((LANES,), jnp.float32)
      for r in range(LANES):
        row = g * LANES + r
        acc = trows_v[row, pl.ds(0, LANES)] * nrows_v[row, pl.ds(0, LANES)]
        for k in range(1, EMBED // LANES):
          acc = acc + (trows_v[row, pl.ds(k * LANES, LANES)] *
                       nrows_v[row, pl.ds(k * LANES, LANES)])
        dots = jnp.where(lane == r, jnp.sum(acc), dots)
      xv = dots + tb_v[pl.ds(g * LANES, LANES)] + nb_v[pl.ds(g * LANES, LANES)]
      yv = 1.0 / (1.0 + jnp.exp(-xv))
      out_v[pl.ds(c * CHUNK + g * LANES, LANES)] = yv

  pltpu.sync_copy(out_v, out_hbm.at[pl.ds(base, BPW)])


@jax.jit
def _run(tidx, nidx, temb, nemb, tb, nb):
  mesh = plsc.VectorSubcoreMesh(core_axis_name="c", subcore_axis_name="s")
  kern = pl.kernel(
      _dot_sigmoid_kernel,
      out_type=jax.ShapeDtypeStruct((BATCH,), jnp.float32),
      mesh=mesh,
      scratch_types=[
          pltpu.VMEM((BPW,), jnp.int32),
          pltpu.VMEM((BPW,), jnp.int32),
          pltpu.VMEM((CHUNK, EMBED), jnp.float32),
          pltpu.VMEM((CHUNK, EMBED), jnp.float32),
          pltpu.VMEM((CHUNK,), jnp.float32),
          pltpu.VMEM((CHUNK,), jnp.float32),
          pltpu.VMEM((BPW,), jnp.float32),
          pltpu.SemaphoreType.DMA,
      ],
  )
  return kern(tidx, nidx, temb, nemb, tb, nb)


def kernel(inputs, track_embedding, name_embedding, track_bias, name_bias):
  tidx = inputs[:, 0].astype(jnp.int32)
  nidx = inputs[:, 1].astype(jnp.int32)
  tb = track_bias.reshape(-1)
  nb = name_bias.reshape(-1)
  return _run(tidx, nidx, track_embedding, name_embedding, tb, nb)


# speedup vs baseline: 1.2081x; 1.2081x over previous
import dataclasses
import functools

import jax
import jax.numpy as jnp
from jax import lax
from jax.experimental import pallas as pl
from jax.experimental.pallas import tpu as pltpu
from jax.experimental.pallas import tpu_sc as plsc

NUM_CORES = 2
NUM_SUBCORES = 16
LANES = 16
NUM_WORKERS = NUM_CORES * NUM_SUBCORES

BATCH = 16384
EMBED = 128
BPW = BATCH // NUM_WORKERS
CHUNK = 256
NCHUNKS = BPW // CHUNK


def _dot_sigmoid_kernel(tidx_hbm, nidx_hbm, temb_hbm, nemb_hbm,
                        tb_hbm, nb_hbm, out_hbm,
                        tidx_v, nidx_v, trows_v, nrows_v,
                        tb_v, nb_v, out_v, sem):
  wid = lax.axis_index("s") * NUM_CORES + lax.axis_index("c")
  base = wid * BPW

  pltpu.sync_copy(tidx_hbm.at[pl.ds(base, BPW)], tidx_v)
  pltpu.sync_copy(nidx_hbm.at[pl.ds(base, BPW)], nidx_v)

  lane = lax.iota(jnp.int32, LANES)

  for c in range(NCHUNKS):
    idx_t = tidx_v.at[pl.ds(c * CHUNK, CHUNK)]
    idx_n = nidx_v.at[pl.ds(c * CHUNK, CHUNK)]
    cps = (
        pltpu.async_copy(temb_hbm.at[idx_t], trows_v, sem),
        pltpu.async_copy(nemb_hbm.at[idx_n], nrows_v, sem),
        pltpu.async_copy(tb_hbm.at[idx_t], tb_v, sem),
        pltpu.async_copy(nb_hbm.at[idx_n], nb_v, sem),
    )
    for cp in cps:
      cp.wait()

    @pl.loop(0, CHUNK // LANES)
    def _(g):
      dots = jnp.zeros((LANES,), jnp.float32)
      for r in range(LANES):
        row = g * LANES + r
        acc = trows_v[row, pl.ds(0, LANES)] * nrows_v[row, pl.ds(0, LANES)]
        for k in range(1, EMBED // LANES):
          acc = acc + (trows_v[row, pl.ds(k * LANES, LANES)] *
                       nrows_v[row, pl.ds(k * LANES, LANES)])
        dots = jnp.where(lane == r, jnp.sum(acc), dots)
      xv = dots + tb_v[pl.ds(g * LANES, LANES)] + nb_v[pl.ds(g * LANES, LANES)]
      yv = 1.0 / (1.0 + jnp.exp(-xv))
      out_v[pl.ds(c * CHUNK + g * LANES, LANES)] = yv

  pltpu.sync_copy(out_v, out_hbm.at[pl.ds(base, BPW)])


@jax.jit
def _run(tidx, nidx, temb, nemb, tb, nb):
  mesh = plsc.VectorSubcoreMesh(core_axis_name="c", subcore_axis_name="s")
  cp = pltpu.CompilerParams()
  if "needs_layout_passes" in pltpu.CompilerParams.__dataclass_fields__:
    cp = dataclasses.replace(cp, needs_layout_passes=False)
  kern = pl.kernel(
      _dot_sigmoid_kernel,
      out_type=jax.ShapeDtypeStruct((BATCH,), jnp.float32),
      mesh=mesh,
      scratch_types=[
          pltpu.VMEM((BPW,), jnp.int32),
          pltpu.VMEM((BPW,), jnp.int32),
          pltpu.VMEM((CHUNK, EMBED), jnp.float32),
          pltpu.VMEM((CHUNK, EMBED), jnp.float32),
          pltpu.VMEM((CHUNK,), jnp.float32),
          pltpu.VMEM((CHUNK,), jnp.float32),
          pltpu.VMEM((BPW,), jnp.float32),
          pltpu.SemaphoreType.DMA,
      ],
      compiler_params=cp,
  )
  return kern(tidx, nidx, temb, nemb, tb, nb)


def kernel(inputs, track_embedding, name_embedding, track_bias, name_bias):
  tidx = inputs[:, 0].astype(jnp.int32)
  nidx = inputs[:, 1].astype(jnp.int32)
  tb = track_bias.reshape(-1)
  nb = name_bias.reshape(-1)
  return _run(tidx, nidx, track_embedding, name_embedding, tb, nb)
